# baseline (speedup 1.0000x reference)
import functools

import jax
import jax.numpy as jnp
from jax import lax
from jax.experimental import pallas as pl
from jax.experimental.pallas import tpu as pltpu
from jax.experimental.pallas import tpu_sc as plsc

N = 10000
E = 160000
BLK = 1000
NB = N // BLK
CH = 80
NCHUNK = E // CH
NSUB = 16
CPT = NCHUNK // NSUB
RPT = 640
NPAD = NSUB * RPT
RLAST = N - 15 * RPT
CHP = 80
NCP = 128
EPAD = NSUB * NCP * CHP
DUMMY = 10008

_HI = lax.Precision.DEFAULT


def _dot(a, b):
    return jnp.dot(a, b, precision=_HI, preferred_element_type=jnp.float32)


def _gelu(x):
    return jax.nn.gelu(x)


def _pre_body(x_ref, pos_ref, lng_ref, lnb_ref,
              pw0_ref, pb0_ref, pw1_ref, pb1_ref, pw2_ref, pb2_ref,
              pw3_ref, pb3_ref, iw0a_ref, iw0b_ref, ib0_ref, iw1_ref, ib1_ref,
              out_ref):
    xb = x_ref[...]
    mu = jnp.mean(xb, axis=1, keepdims=True)
    xc = xb - mu
    va = jnp.mean(xc * xc, axis=1, keepdims=True)
    h = xc / jnp.sqrt(va + 1e-6) * lng_ref[...] + lnb_ref[...]
    pe = pos_ref[...]
    pe = _gelu(_dot(pe, pw0_ref[...]) + pb0_ref[...])
    pe = _gelu(_dot(pe, pw1_ref[...]) + pb1_ref[...])
    pe = _gelu(_dot(pe, pw2_ref[...]) + pb2_ref[...])
    pe = _dot(pe, pw3_ref[...]) + pb3_ref[...]
    t = _gelu(_dot(h, iw0a_ref[...]) + _dot(pe, iw0b_ref[...]) + ib0_ref[...])
    h1 = _gelu(_dot(t, iw1_ref[...]) + ib1_ref[...])
    out_ref[0] = h1[:, :128]
    out_ref[1] = h1[:, 128:]


def _full(shape):
    nd = len(shape)
    return pl.BlockSpec(shape, lambda i, _nd=nd: (0,) * _nd)


def _pre(x, pos, lng, lnb, pws, pbs, iw0a, iw0b, ib0, iw1, ib1):
    specs = [
        pl.BlockSpec((BLK, 256), lambda i: (i, 0)),
        pl.BlockSpec((BLK, 256), lambda i: (i, 0)),
        _full((1, 256)), _full((1, 256)),
        _full((256, 256)), _full((1, 256)),
        _full((256, 256)), _full((1, 256)),
        _full((256, 256)), _full((1, 256)),
        _full((256, 64)), _full((1, 64)),
        _full((256, 512)), _full((64, 512)), _full((1, 512)),
        _full((512, 256)), _full((1, 256)),
    ]
    return pl.pallas_call(
        _pre_body,
        grid=(NB,),
        in_specs=specs,
        out_specs=pl.BlockSpec((2, BLK, 128), lambda i: (0, i, 0)),
        out_shape=jax.ShapeDtypeStruct((2, N, 128), jnp.float32),
    )(x, pos, lng, lnb, pws[0], pbs[0], pws[1], pbs[1], pws[2], pbs[2],
      pws[3], pbs[3], iw0a, iw0b, ib0, iw1, ib1)


def _comb_body(hs_ref, sbb_ref, sct_ref, cb_ref, cc_ref,
               wlb0_ref, wlb1_ref, wrb0_ref, wrb1_ref, bb_ref,
               wlc0_ref, wlc1_ref, wrc0_ref, wrc1_ref, bc_ref,
               out_ref, st_ref):
    h0 = hs_ref[0]
    h1 = hs_ref[1]
    nb = jnp.maximum(cb_ref[...][:, 0:1], 1.0)
    nc = jnp.maximum(cc_ref[...][:, 0:1], 1.0)
    mb0 = sbb_ref[0] / nb
    mb1 = sbb_ref[1] / nb
    mc0 = sct_ref[0] / nc
    mc1 = sct_ref[1] / nc
    hraw = (_dot(h0, wlb0_ref[...]) + _dot(h1, wlb1_ref[...])
            + _dot(mb0, wrb0_ref[...]) + _dot(mb1, wrb1_ref[...]) + bb_ref[...]
            + _dot(h0, wlc0_ref[...]) + _dot(h1, wlc1_ref[...])
            + _dot(mc0, wrc0_ref[...]) + _dot(mc1, wrc1_ref[...]) + bc_ref[...])
    out_ref[...] = hraw

    @pl.when(pl.program_id(0) == 0)
    def _():
        st_ref[...] = jnp.zeros((8, 256), jnp.float32)

    st_ref[0:1, :] += jnp.sum(hraw, axis=0, keepdims=True)
    st_ref[1:2, :] += jnp.sum(hraw * hraw, axis=0, keepdims=True)


def _combine(hs, sbb, sct, cnt_bb, cnt_ct, wlb0, wlb1, wrb0, wrb1, bb,
             wlc0, wlc1, wrc0, wrc1, bc):
    specs = [
        pl.BlockSpec((2, BLK, 128), lambda i: (0, i, 0)),
        pl.BlockSpec((2, BLK, 128), lambda i: (0, i, 0)),
        pl.BlockSpec((2, BLK, 128), lambda i: (0, i, 0)),
        pl.BlockSpec((BLK, 16), lambda i: (i, 0)),
        pl.BlockSpec((BLK, 16), lambda i: (i, 0)),
        _full((128, 256)), _full((128, 256)),
        _full((128, 256)), _full((128, 256)), _full((1, 256)),
        _full((128, 256)), _full((128, 256)),
        _full((128, 256)), _full((128, 256)), _full((1, 256)),
    ]
    return pl.pallas_call(
        _comb_body,
        grid=(NB,),
        in_specs=specs,
        out_specs=[
            pl.BlockSpec((BLK, 256), lambda i: (i, 0)),
            pl.BlockSpec((8, 256), lambda i: (0, 0)),
        ],
        out_shape=[
            jax.ShapeDtypeStruct((N, 256), jnp.float32),
            jax.ShapeDtypeStruct((8, 256), jnp.float32),
        ],
    )(hs, sbb, sct, cnt_bb, cnt_ct, wlb0, wlb1, wrb0, wrb1, bb,
      wlc0, wlc1, wrc0, wrc1, bc)


def _gn_body(hr_ref, st_ref, g_ref, b_ref, a_ref, out_ref):
    a = a_ref[...]
    mu2 = st_ref[0:1, :] * (1.0 / N)
    eh2 = st_ref[1:2, :] * (1.0 / N)
    va2 = eh2 - (2.0 * a - a * a) * mu2 * mu2
    sub = hr_ref[...] - a * mu2
    h = _gelu(g_ref[...] * sub / jnp.sqrt(va2 + 1e-5) + b_ref[...])
    out_ref[0] = h[:, :128]
    out_ref[1] = h[:, 128:]


def _gnorm(hraw, st, g, b, a):
    return pl.pallas_call(
        _gn_body,
        grid=(NB,),
        in_specs=[
            pl.BlockSpec((BLK, 256), lambda i: (i, 0)),
            _full((8, 256)), _full((1, 256)), _full((1, 256)), _full((1, 256)),
        ],
        out_specs=pl.BlockSpec((2, BLK, 128), lambda i: (0, i, 0)),
        out_shape=jax.ShapeDtypeStruct((2, N, 128), jnp.float32),
    )(hraw, st, g, b, a)


def _head_body(h1s_ref, hr_ref, st_ref, g_ref, b_ref, a_ref,
               hw0a0_ref, hw0a1_ref, hw0b_ref, hb0_ref, hw1_ref, hb1_ref,
               aa_ref, ow0a_ref, ow0b_ref, ob0_ref, ow1_ref, ob1_ref,
               ow2_ref, ob2_ref, cbv_ref, out_ref):
    a = a_ref[...]
    mu2 = st_ref[0:1, :] * (1.0 / N)
    eh2 = st_ref[1:2, :] * (1.0 / N)
    va2 = eh2 - (2.0 * a - a * a) * mu2 * mu2
    sub = hr_ref[...] - a * mu2
    h2 = _gelu(g_ref[...] * sub / jnp.sqrt(va2 + 1e-5) + b_ref[...])
    t = _gelu(_dot(h1s_ref[0], hw0a0_ref[...]) + _dot(h1s_ref[1], hw0a1_ref[...])
              + _dot(h2, hw0b_ref[...]) + hb0_ref[...])
    t = _gelu(_dot(t, hw1_ref[...]) + hb1_ref[...])
    u = _gelu(_dot(t, ow0a_ref[...]) + _dot(aa_ref[...], ow0b_ref[...])
              + ob0_ref[...])
    u = _gelu(_dot(u, ow1_ref[...]) + ob1_ref[...])
    z = jnp.tanh(_dot(u, ow2_ref[...]) + ob2_ref[...])
    cbt = cbv_ref[...]
    cc = jnp.sum(cbt * cbt, axis=0, keepdims=True)
    scores = _dot(z, cbt)
    d = cc - 2.0 * scores
    m = jnp.min(d, axis=1, keepdims=True)
    kidx = lax.broadcasted_iota(jnp.int32, d.shape, 1)
    idx = jnp.min(jnp.where(d == m, kidx, 512), axis=1, keepdims=True)
    oh = (kidx == idx).astype(jnp.float32)
    q = lax.dot_general(oh, cbt, (((1,), (1,)), ((), ())),
                        precision=_HI, preferred_element_type=jnp.float32)
    out_ref[...] = z + (q - z)


def _head(h1s, hraw2, st2, g, b, a, hw0a0, hw0a1, hw0b, hb0, hw1, hb1,
          aa, ow0a, ow0b, ob0, ow1, ob1, ow2, ob2, cbv):
    return pl.pallas_call(
        _head_body,
        grid=(NB,),
        in_specs=[
            pl.BlockSpec((2, BLK, 128), lambda i: (0, i, 0)),
            pl.BlockSpec((BLK, 256), lambda i: (i, 0)),
            _full((8, 256)), _full((1, 256)), _full((1, 256)), _full((1, 256)),
            _full((128, 100)), _full((128, 100)), _full((256, 100)),
            _full((1, 100)), _full((100, 100)), _full((1, 100)),
            pl.BlockSpec((BLK, 20), lambda i: (i, 0)),
            _full((100, 100)), _full((20, 100)), _full((1, 100)),
            _full((100, 100)), _full((1, 100)),
            _full((100, 128)), _full((1, 128)),
            _full((128, 512)),
        ],
        out_specs=pl.BlockSpec((BLK, 128), lambda i: (i, 0)),
        out_shape=jax.ShapeDtypeStruct((N, 128), jnp.float32),
    )(h1s, hraw2, st2, g, b, a, hw0a0, hw0a1, hw0b, hb0, hw1, hb1,
      aa, ow0a, ow0b, ob0, ow1, ob1, ow2, ob2, cbv)


def _counts(dst_all, z16):
    mesh = plsc.VectorSubcoreMesh(core_axis_name="c", subcore_axis_name="s")

    @functools.partial(
        pl.kernel, mesh=mesh,
        out_type=jax.ShapeDtypeStruct((2, N, 16), jnp.float32),
        scratch_types=[
            pltpu.VMEM((CPT, CH), jnp.int32),
            pltpu.VMEM((CH, 16), jnp.float32),
            pltpu.VMEM_SHARED((NPAD, 16), jnp.float32),
            pltpu.SemaphoreType.DMA,
        ])
    def k(dst_hbm, zer_hbm, out_hbm, idx_v, ones_v, acc_sh, sem):
        cid = lax.axis_index("c")
        sid = lax.axis_index("s")

        @pl.loop(0, CH)
        def _(i):
            ones_v[i, :] = jnp.full((16,), 1.0, jnp.float32)

        pltpu.sync_copy(zer_hbm.at[sid], acc_sh.at[pl.ds(sid * RPT, RPT)])
        pltpu.sync_copy(dst_hbm.at[cid].at[sid], idx_v)
        plsc.subcore_barrier()

        @pl.loop(0, CPT)
        def _(j):
            pltpu.sync_copy(ones_v, acc_sh.at[idx_v.at[j]], add=True)

        plsc.subcore_barrier()

        @pl.when(sid < NSUB - 1)
        def _():
            pltpu.sync_copy(acc_sh.at[pl.ds(sid * RPT, RPT)],
                            out_hbm.at[cid].at[pl.ds(sid * RPT, RPT)])

        @pl.when(sid == NSUB - 1)
        def _():
            pltpu.sync_copy(acc_sh.at[pl.ds((NSUB - 1) * RPT, RLAST)],
                            out_hbm.at[cid].at[pl.ds((NSUB - 1) * RPT, RLAST)])

    return k(dst_all, z16)


def _segsum(hs, srcf, dst3, z128):
    mesh = plsc.VectorSubcoreMesh(core_axis_name="c", subcore_axis_name="s")

    @functools.partial(
        pl.kernel, mesh=mesh,
        out_type=jax.ShapeDtypeStruct((2, N, 128), jnp.float32),
        scratch_types=[
            pltpu.VMEM((NCP, CHP), jnp.int32),
            pltpu.VMEM((NCP, CHP), jnp.int32),
            pltpu.VMEM((CHP, 128), jnp.float32),
            pltpu.VMEM_SHARED((NPAD, 128), jnp.float32),
            pltpu.SemaphoreType.DMA,
        ])
    def k(hs_hbm, src_hbm, dst_hbm, zer_hbm, out_hbm,
          idxs_v, idxd_v, rows_v, acc_sh, sem):
        cid = lax.axis_index("c")
        sid = lax.axis_index("s")
        pltpu.sync_copy(zer_hbm.at[sid], acc_sh.at[pl.ds(sid * RPT, RPT)])
        pltpu.sync_copy(src_hbm.at[sid], idxs_v)
        pltpu.sync_copy(dst_hbm.at[sid], idxd_v)
        plsc.subcore_barrier()

        @pl.loop(0, NCP)
        def _(j):
            pltpu.async_copy(hs_hbm.at[cid].at[idxs_v.at[j]], rows_v,
                             sem).wait()
            pltpu.sync_copy(rows_v, acc_sh.at[idxd_v.at[j]], add=True)

        plsc.subcore_barrier()

        @pl.when(sid < NSUB - 1)
        def _():
            pltpu.sync_copy(acc_sh.at[pl.ds(sid * RPT, RPT)],
                            out_hbm.at[cid].at[pl.ds(sid * RPT, RPT)])

        @pl.when(sid == NSUB - 1)
        def _():
            pltpu.sync_copy(acc_sh.at[pl.ds((NSUB - 1) * RPT, RLAST)],
                            out_hbm.at[cid].at[pl.ds((NSUB - 1) * RPT, RLAST)])

    return k(hs, srcf, dst3, z128)


def kernel(x, pos, aa, ei_bb, ei_ct, ln_g, ln_b,
           pw0, pb0, pw1, pb1, pw2, pb2, pw3, pb3,
           iw0, ib0, iw1, ib1,
           c0bb_wl, c0bb_wr, c0bb_b, c0ct_wl, c0ct_wr, c0ct_b,
           gn0_g, gn0_b, gn0_a,
           c1bb_wl, c1bb_wr, c1bb_b, c1ct_wl, c1ct_wr, c1ct_b,
           gn1_g, gn1_b, gn1_a,
           hw0, hb0, hw1, hb1,
           ow0, ob0, ow1, ob1, ow2, ob2, cb):
    r1 = lambda v: v.reshape(1, -1)
    spad = jnp.zeros((EPAD - E,), jnp.int32)
    dpad = N + jnp.arange(EPAD - E, dtype=jnp.int32) % (NPAD - N)
    src_bb = jnp.concatenate([ei_bb[0], spad]).reshape(NSUB, NCP, CHP)
    dst_bb = jnp.concatenate([ei_bb[1], dpad]).reshape(NSUB, NCP, CHP)
    src_ct = jnp.concatenate([ei_ct[0], spad]).reshape(NSUB, NCP, CHP)
    dst_ct = jnp.concatenate([ei_ct[1], dpad]).reshape(NSUB, NCP, CHP)
    dst_all = jnp.stack([ei_bb[1].reshape(NSUB, CPT, CH),
                         ei_ct[1].reshape(NSUB, CPT, CH)])
    z16 = jnp.zeros((NSUB, RPT, 16), jnp.float32)
    z128 = jnp.zeros((NSUB, RPT, 128), jnp.float32)

    cnts = _counts(dst_all, z16)
    cnt_bb = cnts[0]
    cnt_ct = cnts[1]

    h0s = _pre(x, pos, r1(ln_g), r1(ln_b),
               [pw0, pw1, pw2, pw3], [r1(pb0), r1(pb1), r1(pb2), r1(pb3)],
               iw0[:256], iw0[256:], r1(ib0), iw1, r1(ib1))

    s1bb = _segsum(h0s, src_bb, dst_bb, z128)
    s1ct = _segsum(h0s, src_ct, dst_ct, z128)
    hraw1, st1 = _combine(h0s, s1bb, s1ct, cnt_bb, cnt_ct,
                          c0bb_wl[:128], c0bb_wl[128:],
                          c0bb_wr[:128], c0bb_wr[128:], r1(c0bb_b),
                          c0ct_wl[:128], c0ct_wl[128:],
                          c0ct_wr[:128], c0ct_wr[128:], r1(c0ct_b))
    h1s = _gnorm(hraw1, st1, r1(gn0_g), r1(gn0_b), r1(gn0_a))

    s2bb = _segsum(h1s, src_bb, dst_bb, z128)
    s2ct = _segsum(h1s, src_ct, dst_ct, z128)
    hraw2, st2 = _combine(h1s, s2bb, s2ct, cnt_bb, cnt_ct,
                          c1bb_wl[:128], c1bb_wl[128:],
                          c1bb_wr[:128], c1bb_wr[128:], r1(c1bb_b),
                          c1ct_wl[:128], c1ct_wl[128:],
                          c1ct_wr[:128], c1ct_wr[128:], r1(c1ct_b))

    return _head(h1s, hraw2, st2, r1(gn1_g), r1(gn1_b), r1(gn1_a),
                 hw0[:128], hw0[128:256], hw0[256:], r1(hb0), hw1, r1(hb1),
                 aa, ow0[:100], ow0[100:], r1(ob0), ow1, r1(ob1),
                 ow2, r1(ob2), cb.T)


# speedup vs baseline: 1.7255x; 1.7255x over previous
import functools

import jax
import jax.numpy as jnp
from jax import lax
from jax.experimental import pallas as pl
from jax.experimental.pallas import tpu as pltpu
from jax.experimental.pallas import tpu_sc as plsc

N = 10000
E = 160000
BLK = 1000
NB = N // BLK
CH = 80
NCHUNK = E // CH
NSUB = 16
CPT = NCHUNK // NSUB
RPT = 640
NPAD = NSUB * RPT
RLAST = N - 15 * RPT
CHP = 80
NCP = 128
EPAD = NSUB * NCP * CHP
DUMMY = 10008

_HI = lax.Precision.DEFAULT


def _dot(a, b):
    return jnp.dot(a, b, precision=_HI, preferred_element_type=jnp.float32)


def _gelu(x):
    return jax.nn.gelu(x)


def _pre_body(x_ref, pos_ref, lng_ref, lnb_ref,
              pw0_ref, pb0_ref, pw1_ref, pb1_ref, pw2_ref, pb2_ref,
              pw3_ref, pb3_ref, iw0a_ref, iw0b_ref, ib0_ref, iw1_ref, ib1_ref,
              out_ref):
    xb = x_ref[...]
    mu = jnp.mean(xb, axis=1, keepdims=True)
    xc = xb - mu
    va = jnp.mean(xc * xc, axis=1, keepdims=True)
    h = xc / jnp.sqrt(va + 1e-6) * lng_ref[...] + lnb_ref[...]
    pe = pos_ref[...]
    pe = _gelu(_dot(pe, pw0_ref[...]) + pb0_ref[...])
    pe = _gelu(_dot(pe, pw1_ref[...]) + pb1_ref[...])
    pe = _gelu(_dot(pe, pw2_ref[...]) + pb2_ref[...])
    pe = _dot(pe, pw3_ref[...]) + pb3_ref[...]
    t = _gelu(_dot(h, iw0a_ref[...]) + _dot(pe, iw0b_ref[...]) + ib0_ref[...])
    h1 = _gelu(_dot(t, iw1_ref[...]) + ib1_ref[...])
    out_ref[0] = h1[:, :128]
    out_ref[1] = h1[:, 128:]


def _full(shape):
    nd = len(shape)
    return pl.BlockSpec(shape, lambda i, _nd=nd: (0,) * _nd)


def _pre(x, pos, lng, lnb, pws, pbs, iw0a, iw0b, ib0, iw1, ib1):
    specs = [
        pl.BlockSpec((BLK, 256), lambda i: (i, 0)),
        pl.BlockSpec((BLK, 256), lambda i: (i, 0)),
        _full((1, 256)), _full((1, 256)),
        _full((256, 256)), _full((1, 256)),
        _full((256, 256)), _full((1, 256)),
        _full((256, 256)), _full((1, 256)),
        _full((256, 64)), _full((1, 64)),
        _full((256, 512)), _full((64, 512)), _full((1, 512)),
        _full((512, 256)), _full((1, 256)),
    ]
    return pl.pallas_call(
        _pre_body,
        grid=(NB,),
        in_specs=specs,
        out_specs=pl.BlockSpec((2, BLK, 128), lambda i: (0, i, 0)),
        out_shape=jax.ShapeDtypeStruct((2, N, 128), jnp.float32),
    )(x, pos, lng, lnb, pws[0], pbs[0], pws[1], pbs[1], pws[2], pbs[2],
      pws[3], pbs[3], iw0a, iw0b, ib0, iw1, ib1)


def _comb_body(hs_ref, sbb_ref, sct_ref, cb_ref, cc_ref,
               wlb0_ref, wlb1_ref, wrb0_ref, wrb1_ref, bb_ref,
               wlc0_ref, wlc1_ref, wrc0_ref, wrc1_ref, bc_ref,
               out_ref, st_ref):
    h0 = hs_ref[0]
    h1 = hs_ref[1]
    nb = jnp.maximum(cb_ref[...][:, 0:1], 1.0)
    nc = jnp.maximum(cc_ref[...][:, 0:1], 1.0)
    mb0 = sbb_ref[0] / nb
    mb1 = sbb_ref[1] / nb
    mc0 = sct_ref[0] / nc
    mc1 = sct_ref[1] / nc
    hraw = (_dot(h0, wlb0_ref[...]) + _dot(h1, wlb1_ref[...])
            + _dot(mb0, wrb0_ref[...]) + _dot(mb1, wrb1_ref[...]) + bb_ref[...]
            + _dot(h0, wlc0_ref[...]) + _dot(h1, wlc1_ref[...])
            + _dot(mc0, wrc0_ref[...]) + _dot(mc1, wrc1_ref[...]) + bc_ref[...])
    out_ref[...] = hraw

    @pl.when(pl.program_id(0) == 0)
    def _():
        st_ref[...] = jnp.zeros((8, 256), jnp.float32)

    st_ref[0:1, :] += jnp.sum(hraw, axis=0, keepdims=True)
    st_ref[1:2, :] += jnp.sum(hraw * hraw, axis=0, keepdims=True)


def _combine(hs, sbb, sct, cnt_bb, cnt_ct, wlb0, wlb1, wrb0, wrb1, bb,
             wlc0, wlc1, wrc0, wrc1, bc):
    specs = [
        pl.BlockSpec((2, BLK, 128), lambda i: (0, i, 0)),
        pl.BlockSpec((2, BLK, 128), lambda i: (0, i, 0)),
        pl.BlockSpec((2, BLK, 128), lambda i: (0, i, 0)),
        pl.BlockSpec((BLK, 16), lambda i: (i, 0)),
        pl.BlockSpec((BLK, 16), lambda i: (i, 0)),
        _full((128, 256)), _full((128, 256)),
        _full((128, 256)), _full((128, 256)), _full((1, 256)),
        _full((128, 256)), _full((128, 256)),
        _full((128, 256)), _full((128, 256)), _full((1, 256)),
    ]
    return pl.pallas_call(
        _comb_body,
        grid=(NB,),
        in_specs=specs,
        out_specs=[
            pl.BlockSpec((BLK, 256), lambda i: (i, 0)),
            pl.BlockSpec((8, 256), lambda i: (0, 0)),
        ],
        out_shape=[
            jax.ShapeDtypeStruct((N, 256), jnp.float32),
            jax.ShapeDtypeStruct((8, 256), jnp.float32),
        ],
    )(hs, sbb, sct, cnt_bb, cnt_ct, wlb0, wlb1, wrb0, wrb1, bb,
      wlc0, wlc1, wrc0, wrc1, bc)


def _gn_body(hr_ref, st_ref, g_ref, b_ref, a_ref, out_ref):
    a = a_ref[...]
    mu2 = st_ref[0:1, :] * (1.0 / N)
    eh2 = st_ref[1:2, :] * (1.0 / N)
    va2 = eh2 - (2.0 * a - a * a) * mu2 * mu2
    sub = hr_ref[...] - a * mu2
    h = _gelu(g_ref[...] * sub / jnp.sqrt(va2 + 1e-5) + b_ref[...])
    out_ref[0] = h[:, :128]
    out_ref[1] = h[:, 128:]


def _gnorm(hraw, st, g, b, a):
    return pl.pallas_call(
        _gn_body,
        grid=(NB,),
        in_specs=[
            pl.BlockSpec((BLK, 256), lambda i: (i, 0)),
            _full((8, 256)), _full((1, 256)), _full((1, 256)), _full((1, 256)),
        ],
        out_specs=pl.BlockSpec((2, BLK, 128), lambda i: (0, i, 0)),
        out_shape=jax.ShapeDtypeStruct((2, N, 128), jnp.float32),
    )(hraw, st, g, b, a)


def _head_body(h1s_ref, hr_ref, st_ref, g_ref, b_ref, a_ref,
               hw0a0_ref, hw0a1_ref, hw0b_ref, hb0_ref, hw1_ref, hb1_ref,
               aa_ref, ow0a_ref, ow0b_ref, ob0_ref, ow1_ref, ob1_ref,
               ow2_ref, ob2_ref, cbv_ref, out_ref):
    a = a_ref[...]
    mu2 = st_ref[0:1, :] * (1.0 / N)
    eh2 = st_ref[1:2, :] * (1.0 / N)
    va2 = eh2 - (2.0 * a - a * a) * mu2 * mu2
    sub = hr_ref[...] - a * mu2
    h2 = _gelu(g_ref[...] * sub / jnp.sqrt(va2 + 1e-5) + b_ref[...])
    t = _gelu(_dot(h1s_ref[0], hw0a0_ref[...]) + _dot(h1s_ref[1], hw0a1_ref[...])
              + _dot(h2, hw0b_ref[...]) + hb0_ref[...])
    t = _gelu(_dot(t, hw1_ref[...]) + hb1_ref[...])
    u = _gelu(_dot(t, ow0a_ref[...]) + _dot(aa_ref[...], ow0b_ref[...])
              + ob0_ref[...])
    u = _gelu(_dot(u, ow1_ref[...]) + ob1_ref[...])
    z = jnp.tanh(_dot(u, ow2_ref[...]) + ob2_ref[...])
    cbt = cbv_ref[...]
    cc = jnp.sum(cbt * cbt, axis=0, keepdims=True)
    scores = _dot(z, cbt)
    d = cc - 2.0 * scores
    m = jnp.min(d, axis=1, keepdims=True)
    kidx = lax.broadcasted_iota(jnp.int32, d.shape, 1)
    idx = jnp.min(jnp.where(d == m, kidx, 512), axis=1, keepdims=True)
    oh = (kidx == idx).astype(jnp.float32)
    q = lax.dot_general(oh, cbt, (((1,), (1,)), ((), ())),
                        precision=_HI, preferred_element_type=jnp.float32)
    out_ref[...] = z + (q - z)


def _head(h1s, hraw2, st2, g, b, a, hw0a0, hw0a1, hw0b, hb0, hw1, hb1,
          aa, ow0a, ow0b, ob0, ow1, ob1, ow2, ob2, cbv):
    return pl.pallas_call(
        _head_body,
        grid=(NB,),
        in_specs=[
            pl.BlockSpec((2, BLK, 128), lambda i: (0, i, 0)),
            pl.BlockSpec((BLK, 256), lambda i: (i, 0)),
            _full((8, 256)), _full((1, 256)), _full((1, 256)), _full((1, 256)),
            _full((128, 100)), _full((128, 100)), _full((256, 100)),
            _full((1, 100)), _full((100, 100)), _full((1, 100)),
            pl.BlockSpec((BLK, 20), lambda i: (i, 0)),
            _full((100, 100)), _full((20, 100)), _full((1, 100)),
            _full((100, 100)), _full((1, 100)),
            _full((100, 128)), _full((1, 128)),
            _full((128, 512)),
        ],
        out_specs=pl.BlockSpec((BLK, 128), lambda i: (i, 0)),
        out_shape=jax.ShapeDtypeStruct((N, 128), jnp.float32),
    )(h1s, hraw2, st2, g, b, a, hw0a0, hw0a1, hw0b, hb0, hw1, hb1,
      aa, ow0a, ow0b, ob0, ow1, ob1, ow2, ob2, cbv)


def _counts(dst_all, z16):
    mesh = plsc.VectorSubcoreMesh(core_axis_name="c", subcore_axis_name="s")

    @functools.partial(
        pl.kernel, mesh=mesh,
        out_type=jax.ShapeDtypeStruct((2, N, 16), jnp.float32),
        scratch_types=[
            pltpu.VMEM((CPT, CH), jnp.int32),
            pltpu.VMEM((CH, 16), jnp.float32),
            pltpu.VMEM_SHARED((NPAD, 16), jnp.float32),
            pltpu.SemaphoreType.DMA,
        ])
    def k(dst_hbm, zer_hbm, out_hbm, idx_v, ones_v, acc_sh, sem):
        cid = lax.axis_index("c")
        sid = lax.axis_index("s")

        @pl.loop(0, CH)
        def _(i):
            ones_v[i, :] = jnp.full((16,), 1.0, jnp.float32)

        pltpu.sync_copy(zer_hbm.at[sid], acc_sh.at[pl.ds(sid * RPT, RPT)])
        pltpu.sync_copy(dst_hbm.at[cid].at[sid], idx_v)
        plsc.subcore_barrier()

        @pl.loop(0, CPT)
        def _(j):
            pltpu.sync_copy(ones_v, acc_sh.at[idx_v.at[j]], add=True)

        plsc.subcore_barrier()

        @pl.when(sid < NSUB - 1)
        def _():
            pltpu.sync_copy(acc_sh.at[pl.ds(sid * RPT, RPT)],
                            out_hbm.at[cid].at[pl.ds(sid * RPT, RPT)])

        @pl.when(sid == NSUB - 1)
        def _():
            pltpu.sync_copy(acc_sh.at[pl.ds((NSUB - 1) * RPT, RLAST)],
                            out_hbm.at[cid].at[pl.ds((NSUB - 1) * RPT, RLAST)])

    return k(dst_all, z16)


def _segsum(hs, srcf, dst3, z128):
    mesh = plsc.VectorSubcoreMesh(core_axis_name="c", subcore_axis_name="s")

    @functools.partial(
        pl.kernel, mesh=mesh,
        out_type=jax.ShapeDtypeStruct((2, N, 128), jnp.float32),
        scratch_types=[
            pltpu.VMEM((CPT, CH), jnp.int32),
            pltpu.VMEM((CPT, CH), jnp.int32),
            pltpu.VMEM((CH, 128), jnp.float32),
            pltpu.VMEM_SHARED((NPAD, 128), jnp.float32),
            pltpu.SemaphoreType.DMA,
        ])
    def k(hs_hbm, src_hbm, dst_hbm, zer_hbm, out_hbm,
          idxs_v, idxd_v, rows_v, acc_sh, sem):
        cid = lax.axis_index("c")
        sid = lax.axis_index("s")
        pltpu.sync_copy(zer_hbm.at[sid], acc_sh.at[pl.ds(sid * RPT, RPT)])
        pltpu.sync_copy(src_hbm.at[sid], idxs_v)
        pltpu.sync_copy(dst_hbm.at[sid], idxd_v)
        plsc.subcore_barrier()

        @pl.loop(0, CPT)
        def _(j):
            pltpu.async_copy(hs_hbm.at[cid].at[idxs_v.at[j]], rows_v,
                             sem).wait()
            pltpu.sync_copy(rows_v, acc_sh.at[idxd_v.at[j]], add=True)

        plsc.subcore_barrier()

        @pl.when(sid < NSUB - 1)
        def _():
            pltpu.sync_copy(acc_sh.at[pl.ds(sid * RPT, RPT)],
                            out_hbm.at[cid].at[pl.ds(sid * RPT, RPT)])

        @pl.when(sid == NSUB - 1)
        def _():
            pltpu.sync_copy(acc_sh.at[pl.ds((NSUB - 1) * RPT, RLAST)],
                            out_hbm.at[cid].at[pl.ds((NSUB - 1) * RPT, RLAST)])

    return k(hs, srcf, dst3, z128)


def kernel(x, pos, aa, ei_bb, ei_ct, ln_g, ln_b,
           pw0, pb0, pw1, pb1, pw2, pb2, pw3, pb3,
           iw0, ib0, iw1, ib1,
           c0bb_wl, c0bb_wr, c0bb_b, c0ct_wl, c0ct_wr, c0ct_b,
           gn0_g, gn0_b, gn0_a,
           c1bb_wl, c1bb_wr, c1bb_b, c1ct_wl, c1ct_wr, c1ct_b,
           gn1_g, gn1_b, gn1_a,
           hw0, hb0, hw1, hb1,
           ow0, ob0, ow1, ob1, ow2, ob2, cb):
    r1 = lambda v: v.reshape(1, -1)
    src_bb = ei_bb[0].reshape(NSUB, CPT, CH)
    dst_bb = ei_bb[1].reshape(NSUB, CPT, CH)
    src_ct = ei_ct[0].reshape(NSUB, CPT, CH)
    dst_ct = ei_ct[1].reshape(NSUB, CPT, CH)
    dst_all = jnp.stack([ei_bb[1].reshape(NSUB, CPT, CH),
                         ei_ct[1].reshape(NSUB, CPT, CH)])
    z16 = jnp.zeros((NSUB, RPT, 16), jnp.float32)
    z128 = jnp.zeros((NSUB, RPT, 128), jnp.float32)

    cnts = _counts(dst_all, z16)
    cnt_bb = cnts[0]
    cnt_ct = cnts[1]

    h0s = _pre(x, pos, r1(ln_g), r1(ln_b),
               [pw0, pw1, pw2, pw3], [r1(pb0), r1(pb1), r1(pb2), r1(pb3)],
               iw0[:256], iw0[256:], r1(ib0), iw1, r1(ib1))

    s1bb = _segsum(h0s, src_bb, dst_bb, z128)
    s1ct = _segsum(h0s, src_ct, dst_ct, z128)
    hraw1, st1 = _combine(h0s, s1bb, s1ct, cnt_bb, cnt_ct,
                          c0bb_wl[:128], c0bb_wl[128:],
                          c0bb_wr[:128], c0bb_wr[128:], r1(c0bb_b),
                          c0ct_wl[:128], c0ct_wl[128:],
                          c0ct_wr[:128], c0ct_wr[128:], r1(c0ct_b))
    h1s = _gnorm(hraw1, st1, r1(gn0_g), r1(gn0_b), r1(gn0_a))

    s2bb = _segsum(h1s, src_bb, dst_bb, z128)
    s2ct = _segsum(h1s, src_ct, dst_ct, z128)
    hraw2, st2 = _combine(h1s, s2bb, s2ct, cnt_bb, cnt_ct,
                          c1bb_wl[:128], c1bb_wl[128:],
                          c1bb_wr[:128], c1bb_wr[128:], r1(c1bb_b),
                          c1ct_wl[:128], c1ct_wl[128:],
                          c1ct_wr[:128], c1ct_wr[128:], r1(c1ct_b))

    return _head(h1s, hraw2, st2, r1(gn1_g), r1(gn1_b), r1(gn1_a),
                 hw0[:128], hw0[128:256], hw0[256:], r1(hb0), hw1, r1(hb1),
                 aa, ow0[:100], ow0[100:], r1(ob0), ow1, r1(ob1),
                 ow2, r1(ob2), cb.T)


# speedup vs baseline: 2.6576x; 1.5402x over previous
import functools

import jax
import jax.numpy as jnp
from jax import lax
from jax.experimental import pallas as pl
from jax.experimental.pallas import tpu as pltpu
from jax.experimental.pallas import tpu_sc as plsc

N = 10000
E = 160000
BLK = 1000
NB = N // BLK
CH = 80
NCHUNK = E // CH
NSUB = 16
CPT = NCHUNK // NSUB
RPT = 640
NPAD = NSUB * RPT
RLAST = N - 15 * RPT
CHP = 80
NCP = 128
EPAD = NSUB * NCP * CHP
DUMMY = 10008

_HI = lax.Precision.DEFAULT


def _dot(a, b):
    return jnp.dot(a, b, precision=_HI, preferred_element_type=jnp.float32)


def _gelu(x):
    return jax.nn.gelu(x)


def _pre_body(x_ref, pos_ref, lng_ref, lnb_ref,
              pw0_ref, pb0_ref, pw1_ref, pb1_ref, pw2_ref, pb2_ref,
              pw3_ref, pb3_ref, iw0a_ref, iw0b_ref, ib0_ref, iw1_ref, ib1_ref,
              out_ref):
    xb = x_ref[...]
    mu = jnp.mean(xb, axis=1, keepdims=True)
    xc = xb - mu
    va = jnp.mean(xc * xc, axis=1, keepdims=True)
    h = xc / jnp.sqrt(va + 1e-6) * lng_ref[...] + lnb_ref[...]
    pe = pos_ref[...]
    pe = _gelu(_dot(pe, pw0_ref[...]) + pb0_ref[...])
    pe = _gelu(_dot(pe, pw1_ref[...]) + pb1_ref[...])
    pe = _gelu(_dot(pe, pw2_ref[...]) + pb2_ref[...])
    pe = _dot(pe, pw3_ref[...]) + pb3_ref[...]
    t = _gelu(_dot(h, iw0a_ref[...]) + _dot(pe, iw0b_ref[...]) + ib0_ref[...])
    h1 = _gelu(_dot(t, iw1_ref[...]) + ib1_ref[...])
    out_ref[0] = h1[:, :128]
    out_ref[1] = h1[:, 128:]


def _full(shape):
    nd = len(shape)
    return pl.BlockSpec(shape, lambda i, _nd=nd: (0,) * _nd)


def _pre(x, pos, lng, lnb, pws, pbs, iw0a, iw0b, ib0, iw1, ib1):
    specs = [
        pl.BlockSpec((BLK, 256), lambda i: (i, 0)),
        pl.BlockSpec((BLK, 256), lambda i: (i, 0)),
        _full((1, 256)), _full((1, 256)),
        _full((256, 256)), _full((1, 256)),
        _full((256, 256)), _full((1, 256)),
        _full((256, 256)), _full((1, 256)),
        _full((256, 64)), _full((1, 64)),
        _full((256, 512)), _full((64, 512)), _full((1, 512)),
        _full((512, 256)), _full((1, 256)),
    ]
    return pl.pallas_call(
        _pre_body,
        grid=(NB,),
        in_specs=specs,
        out_specs=pl.BlockSpec((2, BLK, 128), lambda i: (0, i, 0)),
        out_shape=jax.ShapeDtypeStruct((2, N, 128), jnp.float32),
    )(x, pos, lng, lnb, pws[0], pbs[0], pws[1], pbs[1], pws[2], pbs[2],
      pws[3], pbs[3], iw0a, iw0b, ib0, iw1, ib1)


def _comb_body(hs_ref, sbb_ref, sct_ref, cb_ref, cc_ref,
               wlb0_ref, wlb1_ref, wrb0_ref, wrb1_ref, bb_ref,
               wlc0_ref, wlc1_ref, wrc0_ref, wrc1_ref, bc_ref,
               out_ref, st_ref):
    h0 = hs_ref[0]
    h1 = hs_ref[1]
    nb = jnp.maximum(cb_ref[...][:, 0:1], 1.0)
    nc = jnp.maximum(cc_ref[...][:, 0:1], 1.0)
    mb0 = sbb_ref[0] / nb
    mb1 = sbb_ref[1] / nb
    mc0 = sct_ref[0] / nc
    mc1 = sct_ref[1] / nc
    hraw = (_dot(h0, wlb0_ref[...]) + _dot(h1, wlb1_ref[...])
            + _dot(mb0, wrb0_ref[...]) + _dot(mb1, wrb1_ref[...]) + bb_ref[...]
            + _dot(h0, wlc0_ref[...]) + _dot(h1, wlc1_ref[...])
            + _dot(mc0, wrc0_ref[...]) + _dot(mc1, wrc1_ref[...]) + bc_ref[...])
    out_ref[...] = hraw

    @pl.when(pl.program_id(0) == 0)
    def _():
        st_ref[...] = jnp.zeros((8, 256), jnp.float32)

    st_ref[0:1, :] += jnp.sum(hraw, axis=0, keepdims=True)
    st_ref[1:2, :] += jnp.sum(hraw * hraw, axis=0, keepdims=True)


def _combine(hs, sbb, sct, cnt_bb, cnt_ct, wlb0, wlb1, wrb0, wrb1, bb,
             wlc0, wlc1, wrc0, wrc1, bc):
    specs = [
        pl.BlockSpec((2, BLK, 128), lambda i: (0, i, 0)),
        pl.BlockSpec((2, BLK, 128), lambda i: (0, i, 0)),
        pl.BlockSpec((2, BLK, 128), lambda i: (0, i, 0)),
        pl.BlockSpec((BLK, 16), lambda i: (i, 0)),
        pl.BlockSpec((BLK, 16), lambda i: (i, 0)),
        _full((128, 256)), _full((128, 256)),
        _full((128, 256)), _full((128, 256)), _full((1, 256)),
        _full((128, 256)), _full((128, 256)),
        _full((128, 256)), _full((128, 256)), _full((1, 256)),
    ]
    return pl.pallas_call(
        _comb_body,
        grid=(NB,),
        in_specs=specs,
        out_specs=[
            pl.BlockSpec((BLK, 256), lambda i: (i, 0)),
            pl.BlockSpec((8, 256), lambda i: (0, 0)),
        ],
        out_shape=[
            jax.ShapeDtypeStruct((N, 256), jnp.float32),
            jax.ShapeDtypeStruct((8, 256), jnp.float32),
        ],
    )(hs, sbb, sct, cnt_bb, cnt_ct, wlb0, wlb1, wrb0, wrb1, bb,
      wlc0, wlc1, wrc0, wrc1, bc)


def _gn_body(hr_ref, st_ref, g_ref, b_ref, a_ref, out_ref):
    a = a_ref[...]
    mu2 = st_ref[0:1, :] * (1.0 / N)
    eh2 = st_ref[1:2, :] * (1.0 / N)
    va2 = eh2 - (2.0 * a - a * a) * mu2 * mu2
    sub = hr_ref[...] - a * mu2
    h = _gelu(g_ref[...] * sub / jnp.sqrt(va2 + 1e-5) + b_ref[...])
    out_ref[0] = h[:, :128]
    out_ref[1] = h[:, 128:]


def _gnorm(hraw, st, g, b, a):
    return pl.pallas_call(
        _gn_body,
        grid=(NB,),
        in_specs=[
            pl.BlockSpec((BLK, 256), lambda i: (i, 0)),
            _full((8, 256)), _full((1, 256)), _full((1, 256)), _full((1, 256)),
        ],
        out_specs=pl.BlockSpec((2, BLK, 128), lambda i: (0, i, 0)),
        out_shape=jax.ShapeDtypeStruct((2, N, 128), jnp.float32),
    )(hraw, st, g, b, a)


def _head_body(h1s_ref, hr_ref, st_ref, g_ref, b_ref, a_ref,
               hw0a0_ref, hw0a1_ref, hw0b_ref, hb0_ref, hw1_ref, hb1_ref,
               aa_ref, ow0a_ref, ow0b_ref, ob0_ref, ow1_ref, ob1_ref,
               ow2_ref, ob2_ref, cbv_ref, out_ref):
    a = a_ref[...]
    mu2 = st_ref[0:1, :] * (1.0 / N)
    eh2 = st_ref[1:2, :] * (1.0 / N)
    va2 = eh2 - (2.0 * a - a * a) * mu2 * mu2
    sub = hr_ref[...] - a * mu2
    h2 = _gelu(g_ref[...] * sub / jnp.sqrt(va2 + 1e-5) + b_ref[...])
    t = _gelu(_dot(h1s_ref[0], hw0a0_ref[...]) + _dot(h1s_ref[1], hw0a1_ref[...])
              + _dot(h2, hw0b_ref[...]) + hb0_ref[...])
    t = _gelu(_dot(t, hw1_ref[...]) + hb1_ref[...])
    u = _gelu(_dot(t, ow0a_ref[...]) + _dot(aa_ref[...], ow0b_ref[...])
              + ob0_ref[...])
    u = _gelu(_dot(u, ow1_ref[...]) + ob1_ref[...])
    z = jnp.tanh(_dot(u, ow2_ref[...]) + ob2_ref[...])
    cbt = cbv_ref[...]
    cc = jnp.sum(cbt * cbt, axis=0, keepdims=True)
    scores = _dot(z, cbt)
    d = cc - 2.0 * scores
    m = jnp.min(d, axis=1, keepdims=True)
    kidx = lax.broadcasted_iota(jnp.int32, d.shape, 1)
    idx = jnp.min(jnp.where(d == m, kidx, 512), axis=1, keepdims=True)
    oh = (kidx == idx).astype(jnp.float32)
    q = lax.dot_general(oh, cbt, (((1,), (1,)), ((), ())),
                        precision=_HI, preferred_element_type=jnp.float32)
    out_ref[...] = z + (q - z)


def _head(h1s, hraw2, st2, g, b, a, hw0a0, hw0a1, hw0b, hb0, hw1, hb1,
          aa, ow0a, ow0b, ob0, ow1, ob1, ow2, ob2, cbv):
    return pl.pallas_call(
        _head_body,
        grid=(NB,),
        in_specs=[
            pl.BlockSpec((2, BLK, 128), lambda i: (0, i, 0)),
            pl.BlockSpec((BLK, 256), lambda i: (i, 0)),
            _full((8, 256)), _full((1, 256)), _full((1, 256)), _full((1, 256)),
            _full((128, 100)), _full((128, 100)), _full((256, 100)),
            _full((1, 100)), _full((100, 100)), _full((1, 100)),
            pl.BlockSpec((BLK, 20), lambda i: (i, 0)),
            _full((100, 100)), _full((20, 100)), _full((1, 100)),
            _full((100, 100)), _full((1, 100)),
            _full((100, 128)), _full((1, 128)),
            _full((128, 512)),
        ],
        out_specs=pl.BlockSpec((BLK, 128), lambda i: (i, 0)),
        out_shape=jax.ShapeDtypeStruct((N, 128), jnp.float32),
    )(h1s, hraw2, st2, g, b, a, hw0a0, hw0a1, hw0b, hb0, hw1, hb1,
      aa, ow0a, ow0b, ob0, ow1, ob1, ow2, ob2, cbv)


def _counts(dst_all, z16):
    mesh = plsc.VectorSubcoreMesh(core_axis_name="c", subcore_axis_name="s")

    @functools.partial(
        pl.kernel, mesh=mesh,
        out_type=jax.ShapeDtypeStruct((2, N, 16), jnp.float32),
        scratch_types=[
            pltpu.VMEM((CPT, CH), jnp.int32),
            pltpu.VMEM((CH, 16), jnp.float32),
            pltpu.VMEM_SHARED((NPAD, 16), jnp.float32),
            pltpu.SemaphoreType.DMA,
        ])
    def k(dst_hbm, zer_hbm, out_hbm, idx_v, ones_v, acc_sh, sem):
        cid = lax.axis_index("c")
        sid = lax.axis_index("s")

        @pl.loop(0, CH)
        def _(i):
            ones_v[i, :] = jnp.full((16,), 1.0, jnp.float32)

        pltpu.sync_copy(zer_hbm.at[sid], acc_sh.at[pl.ds(sid * RPT, RPT)])
        pltpu.sync_copy(dst_hbm.at[cid].at[sid], idx_v)
        plsc.subcore_barrier()

        @pl.loop(0, CPT)
        def _(j):
            pltpu.sync_copy(ones_v, acc_sh.at[idx_v.at[j]], add=True)

        plsc.subcore_barrier()

        @pl.when(sid < NSUB - 1)
        def _():
            pltpu.sync_copy(acc_sh.at[pl.ds(sid * RPT, RPT)],
                            out_hbm.at[cid].at[pl.ds(sid * RPT, RPT)])

        @pl.when(sid == NSUB - 1)
        def _():
            pltpu.sync_copy(acc_sh.at[pl.ds((NSUB - 1) * RPT, RLAST)],
                            out_hbm.at[cid].at[pl.ds((NSUB - 1) * RPT, RLAST)])

    return k(dst_all, z16)


def _segsum(hs, srcf, dst3, z128):
    mesh = plsc.VectorSubcoreMesh(core_axis_name="c", subcore_axis_name="s")

    @functools.partial(
        pl.kernel, mesh=mesh,
        out_type=jax.ShapeDtypeStruct((2, N, 128), jnp.float32),
        scratch_types=[
            pltpu.VMEM((CPT * CH,), jnp.int32),
            pltpu.VMEM((CPT, CH), jnp.int32),
            pltpu.VMEM((CH, 128), jnp.float32),
            pltpu.VMEM((CH, 128), jnp.float32),
            pltpu.VMEM_SHARED((NPAD, 128), jnp.float32),
            pltpu.SemaphoreType.DMA,
            pltpu.SemaphoreType.DMA,
        ])
    def k(hs_hbm, src_hbm, dst_hbm, zer_hbm, out_hbm,
          idxs_v, idxd_v, bufa_v, bufb_v, acc_sh, sema, semb):
        cid = lax.axis_index("c")
        sid = lax.axis_index("s")
        pltpu.sync_copy(zer_hbm.at[sid], acc_sh.at[pl.ds(sid * RPT, RPT)])
        pltpu.sync_copy(src_hbm.at[sid], idxs_v)
        pltpu.sync_copy(dst_hbm.at[sid], idxd_v)
        plsc.subcore_barrier()

        def gat(j, buf, sem):
            return pltpu.make_async_copy(
                hs_hbm.at[cid].at[idxs_v.at[pl.ds(j * CH, CH)]], buf, sem)

        gat(0, bufa_v, sema).start()

        @pl.loop(0, (CPT - 1) // 2)
        def _(p):
            j0 = 2 * p
            gat(j0 + 1, bufb_v, semb).start()
            gat(j0, bufa_v, sema).wait()
            pltpu.sync_copy(bufa_v, acc_sh.at[idxd_v.at[j0]], add=True)
            gat(j0 + 2, bufa_v, sema).start()
            gat(j0 + 1, bufb_v, semb).wait()
            pltpu.sync_copy(bufb_v, acc_sh.at[idxd_v.at[j0 + 1]], add=True)

        gat(CPT - 1, bufa_v, sema).wait()
        pltpu.sync_copy(bufa_v, acc_sh.at[idxd_v.at[CPT - 1]], add=True)
        plsc.subcore_barrier()

        @pl.when(sid < NSUB - 1)
        def _():
            pltpu.sync_copy(acc_sh.at[pl.ds(sid * RPT, RPT)],
                            out_hbm.at[cid].at[pl.ds(sid * RPT, RPT)])

        @pl.when(sid == NSUB - 1)
        def _():
            pltpu.sync_copy(acc_sh.at[pl.ds((NSUB - 1) * RPT, RLAST)],
                            out_hbm.at[cid].at[pl.ds((NSUB - 1) * RPT, RLAST)])

    return k(hs, srcf, dst3, z128)


def kernel(x, pos, aa, ei_bb, ei_ct, ln_g, ln_b,
           pw0, pb0, pw1, pb1, pw2, pb2, pw3, pb3,
           iw0, ib0, iw1, ib1,
           c0bb_wl, c0bb_wr, c0bb_b, c0ct_wl, c0ct_wr, c0ct_b,
           gn0_g, gn0_b, gn0_a,
           c1bb_wl, c1bb_wr, c1bb_b, c1ct_wl, c1ct_wr, c1ct_b,
           gn1_g, gn1_b, gn1_a,
           hw0, hb0, hw1, hb1,
           ow0, ob0, ow1, ob1, ow2, ob2, cb):
    r1 = lambda v: v.reshape(1, -1)
    src_bb = ei_bb[0].reshape(NSUB, CPT * CH)
    dst_bb = ei_bb[1].reshape(NSUB, CPT, CH)
    src_ct = ei_ct[0].reshape(NSUB, CPT * CH)
    dst_ct = ei_ct[1].reshape(NSUB, CPT, CH)
    dst_all = jnp.stack([ei_bb[1].reshape(NSUB, CPT, CH),
                         ei_ct[1].reshape(NSUB, CPT, CH)])
    z16 = jnp.zeros((NSUB, RPT, 16), jnp.float32)
    z128 = jnp.zeros((NSUB, RPT, 128), jnp.float32)

    cnts = _counts(dst_all, z16)
    cnt_bb = cnts[0]
    cnt_ct = cnts[1]

    h0s = _pre(x, pos, r1(ln_g), r1(ln_b),
               [pw0, pw1, pw2, pw3], [r1(pb0), r1(pb1), r1(pb2), r1(pb3)],
               iw0[:256], iw0[256:], r1(ib0), iw1, r1(ib1))

    s1bb = _segsum(h0s, src_bb, dst_bb, z128)
    s1ct = _segsum(h0s, src_ct, dst_ct, z128)
    hraw1, st1 = _combine(h0s, s1bb, s1ct, cnt_bb, cnt_ct,
                          c0bb_wl[:128], c0bb_wl[128:],
                          c0bb_wr[:128], c0bb_wr[128:], r1(c0bb_b),
                          c0ct_wl[:128], c0ct_wl[128:],
                          c0ct_wr[:128], c0ct_wr[128:], r1(c0ct_b))
    h1s = _gnorm(hraw1, st1, r1(gn0_g), r1(gn0_b), r1(gn0_a))

    s2bb = _segsum(h1s, src_bb, dst_bb, z128)
    s2ct = _segsum(h1s, src_ct, dst_ct, z128)
    hraw2, st2 = _combine(h1s, s2bb, s2ct, cnt_bb, cnt_ct,
                          c1bb_wl[:128], c1bb_wl[128:],
                          c1bb_wr[:128], c1bb_wr[128:], r1(c1bb_b),
                          c1ct_wl[:128], c1ct_wl[128:],
                          c1ct_wr[:128], c1ct_wr[128:], r1(c1ct_b))

    return _head(h1s, hraw2, st2, r1(gn1_g), r1(gn1_b), r1(gn1_a),
                 hw0[:128], hw0[128:256], hw0[256:], r1(hb0), hw1, r1(hb1),
                 aa, ow0[:100], ow0[100:], r1(ob0), ow1, r1(ob1),
                 ow2, r1(ob2), cb.T)


# speedup vs baseline: 2.6577x; 1.0000x over previous
import functools

import jax
import jax.numpy as jnp
from jax import lax
from jax.experimental import pallas as pl
from jax.experimental.pallas import tpu as pltpu
from jax.experimental.pallas import tpu_sc as plsc

N = 10000
E = 160000
BLK = 1000
NB = N // BLK
CH = 80
NCHUNK = E // CH
NSUB = 16
CPT = NCHUNK // NSUB
RPT = 640
NPAD = NSUB * RPT
RLAST = N - 15 * RPT
CHP = 80
NCP = 128
EPAD = NSUB * NCP * CHP
DUMMY = 10008

_HI = lax.Precision.DEFAULT


def _dot(a, b):
    return jnp.dot(a, b, precision=_HI, preferred_element_type=jnp.float32)


def _gelu(x):
    return jax.nn.gelu(x)


def _pre_body(x_ref, pos_ref, lng_ref, lnb_ref,
              pw0_ref, pb0_ref, pw1_ref, pb1_ref, pw2_ref, pb2_ref,
              pw3_ref, pb3_ref, iw0a_ref, iw0b_ref, ib0_ref, iw1_ref, ib1_ref,
              out_ref):
    xb = x_ref[...]
    mu = jnp.mean(xb, axis=1, keepdims=True)
    xc = xb - mu
    va = jnp.mean(xc * xc, axis=1, keepdims=True)
    h = xc / jnp.sqrt(va + 1e-6) * lng_ref[...] + lnb_ref[...]
    pe = pos_ref[...]
    pe = _gelu(_dot(pe, pw0_ref[...]) + pb0_ref[...])
    pe = _gelu(_dot(pe, pw1_ref[...]) + pb1_ref[...])
    pe = _gelu(_dot(pe, pw2_ref[...]) + pb2_ref[...])
    pe = _dot(pe, pw3_ref[...]) + pb3_ref[...]
    t = _gelu(_dot(h, iw0a_ref[...]) + _dot(pe, iw0b_ref[...]) + ib0_ref[...])
    h1 = _gelu(_dot(t, iw1_ref[...]) + ib1_ref[...])
    out_ref[0] = h1[:, :128]
    out_ref[1] = h1[:, 128:]


def _full(shape):
    nd = len(shape)
    return pl.BlockSpec(shape, lambda i, _nd=nd: (0,) * _nd)


def _pre(x, pos, lng, lnb, pws, pbs, iw0a, iw0b, ib0, iw1, ib1):
    specs = [
        pl.BlockSpec((BLK, 256), lambda i: (i, 0)),
        pl.BlockSpec((BLK, 256), lambda i: (i, 0)),
        _full((1, 256)), _full((1, 256)),
        _full((256, 256)), _full((1, 256)),
        _full((256, 256)), _full((1, 256)),
        _full((256, 256)), _full((1, 256)),
        _full((256, 64)), _full((1, 64)),
        _full((256, 512)), _full((64, 512)), _full((1, 512)),
        _full((512, 256)), _full((1, 256)),
    ]
    return pl.pallas_call(
        _pre_body,
        grid=(NB,),
        in_specs=specs,
        out_specs=pl.BlockSpec((2, BLK, 128), lambda i: (0, i, 0)),
        out_shape=jax.ShapeDtypeStruct((2, N, 128), jnp.float32),
    )(x, pos, lng, lnb, pws[0], pbs[0], pws[1], pbs[1], pws[2], pbs[2],
      pws[3], pbs[3], iw0a, iw0b, ib0, iw1, ib1)


def _combine_block(i, hs_ref, sbb_ref, sct_ref, cb_ref, cc_ref, wrefs, hraw_v,
                   st_v):
    (wlb0_ref, wlb1_ref, wrb0_ref, wrb1_ref, bb_ref,
     wlc0_ref, wlc1_ref, wrc0_ref, wrc1_ref, bc_ref) = wrefs
    h0 = hs_ref[0]
    h1 = hs_ref[1]
    nb = jnp.maximum(cb_ref[...][:, 0:1], 1.0)
    nc = jnp.maximum(cc_ref[...][:, 0:1], 1.0)
    mb0 = sbb_ref[0] / nb
    mb1 = sbb_ref[1] / nb
    mc0 = sct_ref[0] / nc
    mc1 = sct_ref[1] / nc
    hraw = (_dot(h0, wlb0_ref[...]) + _dot(h1, wlb1_ref[...])
            + _dot(mb0, wrb0_ref[...]) + _dot(mb1, wrb1_ref[...]) + bb_ref[...]
            + _dot(h0, wlc0_ref[...]) + _dot(h1, wlc1_ref[...])
            + _dot(mc0, wrc0_ref[...]) + _dot(mc1, wrc1_ref[...]) + bc_ref[...])
    hraw_v[pl.ds(i * BLK, BLK), :] = hraw

    @pl.when(i == 0)
    def _():
        st_v[...] = jnp.zeros((8, 256), jnp.float32)

    st_v[0:1, :] += jnp.sum(hraw, axis=0, keepdims=True)
    st_v[1:2, :] += jnp.sum(hraw * hraw, axis=0, keepdims=True)


def _gn_apply(ii, st_v, g_ref, b_ref, a_ref, hraw_v):
    a = a_ref[...]
    mu2 = st_v[0:1, :] * (1.0 / N)
    eh2 = st_v[1:2, :] * (1.0 / N)
    va2 = eh2 - (2.0 * a - a * a) * mu2 * mu2
    sub = hraw_v[pl.ds(ii * BLK, BLK), :] - a * mu2
    return _gelu(g_ref[...] * sub / jnp.sqrt(va2 + 1e-5) + b_ref[...])


def _cg_body(hs_ref, sbb_ref, sct_ref, cb_ref, cc_ref,
             wlb0_ref, wlb1_ref, wrb0_ref, wrb1_ref, bb_ref,
             wlc0_ref, wlc1_ref, wrc0_ref, wrc1_ref, bc_ref,
             g_ref, b_ref, a_ref, out_ref, hraw_v, st_v):
    i = pl.program_id(0)

    @pl.when(i < NB)
    def _():
        _combine_block(i, hs_ref, sbb_ref, sct_ref, cb_ref, cc_ref,
                       (wlb0_ref, wlb1_ref, wrb0_ref, wrb1_ref, bb_ref,
                        wlc0_ref, wlc1_ref, wrc0_ref, wrc1_ref, bc_ref),
                       hraw_v, st_v)

    @pl.when(i >= NB)
    def _():
        h = _gn_apply(i - NB, st_v, g_ref, b_ref, a_ref, hraw_v)
        out_ref[0] = h[:, :128]
        out_ref[1] = h[:, 128:]


def _combine_gn(hs, sbb, sct, cnt_bb, cnt_ct, wlb0, wlb1, wrb0, wrb1, bb,
                wlc0, wlc1, wrc0, wrc1, bc, g, b, a):
    ix = lambda i: (0, jnp.where(i < NB, i, i - NB), 0)
    ir = lambda i: (jnp.where(i < NB, i, i - NB), 0)
    specs = [
        pl.BlockSpec((2, BLK, 128), ix),
        pl.BlockSpec((2, BLK, 128), ix),
        pl.BlockSpec((2, BLK, 128), ix),
        pl.BlockSpec((BLK, 16), ir),
        pl.BlockSpec((BLK, 16), ir),
        _full((128, 256)), _full((128, 256)),
        _full((128, 256)), _full((128, 256)), _full((1, 256)),
        _full((128, 256)), _full((128, 256)),
        _full((128, 256)), _full((128, 256)), _full((1, 256)),
        _full((1, 256)), _full((1, 256)), _full((1, 256)),
    ]
    return pl.pallas_call(
        _cg_body,
        grid=(2 * NB,),
        in_specs=specs,
        out_specs=pl.BlockSpec((2, BLK, 128),
                               lambda i: (0, jnp.maximum(i - NB, 0), 0)),
        out_shape=jax.ShapeDtypeStruct((2, N, 128), jnp.float32),
        scratch_shapes=[pltpu.VMEM((N, 256), jnp.float32),
                        pltpu.VMEM((8, 256), jnp.float32)],
    )(hs, sbb, sct, cnt_bb, cnt_ct, wlb0, wlb1, wrb0, wrb1, bb,
      wlc0, wlc1, wrc0, wrc1, bc, g, b, a)


def _ch_body(h1s_ref, sbb_ref, sct_ref, cb2_ref, cc2_ref,
             wlb0_ref, wlb1_ref, wrb0_ref, wrb1_ref, bb2_ref,
             wlc0_ref, wlc1_ref, wrc0_ref, wrc1_ref, bc2_ref,
             g_ref, b_ref, a_ref,
             hw0a0_ref, hw0a1_ref, hw0b_ref, hb0_ref, hw1_ref, hb1_ref,
             aa_ref, ow0a_ref, ow0b_ref, ob0_ref, ow1_ref, ob1_ref,
             ow2_ref, ob2_ref, cbv_ref, out_ref, hraw_v, st_v):
    i = pl.program_id(0)

    @pl.when(i < NB)
    def _():
        _combine_block(i, h1s_ref, sbb_ref, sct_ref, cb2_ref, cc2_ref,
                       (wlb0_ref, wlb1_ref, wrb0_ref, wrb1_ref, bb2_ref,
                        wlc0_ref, wlc1_ref, wrc0_ref, wrc1_ref, bc2_ref),
                       hraw_v, st_v)

    @pl.when(i >= NB)
    def _():
        h2 = _gn_apply(i - NB, st_v, g_ref, b_ref, a_ref, hraw_v)
        t = _gelu(_dot(h1s_ref[0], hw0a0_ref[...])
                  + _dot(h1s_ref[1], hw0a1_ref[...])
                  + _dot(h2, hw0b_ref[...]) + hb0_ref[...])
        t = _gelu(_dot(t, hw1_ref[...]) + hb1_ref[...])
        u = _gelu(_dot(t, ow0a_ref[...]) + _dot(aa_ref[...], ow0b_ref[...])
                  + ob0_ref[...])
        u = _gelu(_dot(u, ow1_ref[...]) + ob1_ref[...])
        z = jnp.tanh(_dot(u, ow2_ref[...]) + ob2_ref[...])
        cbt = cbv_ref[...]
        cc = jnp.sum(cbt * cbt, axis=0, keepdims=True)
        scores = _dot(z, cbt)
        d = cc - 2.0 * scores
        m = jnp.min(d, axis=1, keepdims=True)
        kidx = lax.broadcasted_iota(jnp.int32, d.shape, 1)
        idx = jnp.min(jnp.where(d == m, kidx, 512), axis=1, keepdims=True)
        oh = (kidx == idx).astype(jnp.float32)
        q = lax.dot_general(oh, cbt, (((1,), (1,)), ((), ())),
                            precision=_HI, preferred_element_type=jnp.float32)
        out_ref[...] = z + (q - z)


def _head(h1s, sbb, sct, cnt_bb, cnt_ct,
          wlb0, wlb1, wrb0, wrb1, bb2, wlc0, wlc1, wrc0, wrc1, bc2,
          g, b, a, hw0a0, hw0a1, hw0b, hb0, hw1, hb1,
          aa, ow0a, ow0b, ob0, ow1, ob1, ow2, ob2, cbv):
    ix = lambda i: (0, jnp.where(i < NB, i, i - NB), 0)
    ir = lambda i: (jnp.where(i < NB, i, i - NB), 0)
    return pl.pallas_call(
        _ch_body,
        grid=(2 * NB,),
        in_specs=[
            pl.BlockSpec((2, BLK, 128), ix),
            pl.BlockSpec((2, BLK, 128), ix),
            pl.BlockSpec((2, BLK, 128), ix),
            pl.BlockSpec((BLK, 16), ir),
            pl.BlockSpec((BLK, 16), ir),
            _full((128, 256)), _full((128, 256)),
            _full((128, 256)), _full((128, 256)), _full((1, 256)),
            _full((128, 256)), _full((128, 256)),
            _full((128, 256)), _full((128, 256)), _full((1, 256)),
            _full((1, 256)), _full((1, 256)), _full((1, 256)),
            _full((128, 100)), _full((128, 100)), _full((256, 100)),
            _full((1, 100)), _full((100, 100)), _full((1, 100)),
            pl.BlockSpec((BLK, 20), ir),
            _full((100, 100)), _full((20, 100)), _full((1, 100)),
            _full((100, 100)), _full((1, 100)),
            _full((100, 128)), _full((1, 128)),
            _full((128, 512)),
        ],
        out_specs=pl.BlockSpec((BLK, 128),
                               lambda i: (jnp.maximum(i - NB, 0), 0)),
        out_shape=jax.ShapeDtypeStruct((N, 128), jnp.float32),
        scratch_shapes=[pltpu.VMEM((N, 256), jnp.float32),
                        pltpu.VMEM((8, 256), jnp.float32)],
    )(h1s, sbb, sct, cnt_bb, cnt_ct,
      wlb0, wlb1, wrb0, wrb1, bb2, wlc0, wlc1, wrc0, wrc1, bc2,
      g, b, a, hw0a0, hw0a1, hw0b, hb0, hw1, hb1,
      aa, ow0a, ow0b, ob0, ow1, ob1, ow2, ob2, cbv)


def _counts(dst_all, z16):
    mesh = plsc.VectorSubcoreMesh(core_axis_name="c", subcore_axis_name="s")

    @functools.partial(
        pl.kernel, mesh=mesh,
        out_type=jax.ShapeDtypeStruct((2, N, 16), jnp.float32),
        scratch_types=[
            pltpu.VMEM((CPT, CH), jnp.int32),
            pltpu.VMEM((CH, 16), jnp.float32),
            pltpu.VMEM_SHARED((NPAD, 16), jnp.float32),
            pltpu.SemaphoreType.DMA,
        ])
    def k(dst_hbm, zer_hbm, out_hbm, idx_v, ones_v, acc_sh, sem):
        cid = lax.axis_index("c")
        sid = lax.axis_index("s")

        @pl.loop(0, CH)
        def _(i):
            ones_v[i, :] = jnp.full((16,), 1.0, jnp.float32)

        pltpu.sync_copy(zer_hbm.at[sid], acc_sh.at[pl.ds(sid * RPT, RPT)])
        pltpu.sync_copy(dst_hbm.at[cid].at[sid], idx_v)
        plsc.subcore_barrier()

        @pl.loop(0, CPT)
        def _(j):
            pltpu.sync_copy(ones_v, acc_sh.at[idx_v.at[j]], add=True)

        plsc.subcore_barrier()

        @pl.when(sid < NSUB - 1)
        def _():
            pltpu.sync_copy(acc_sh.at[pl.ds(sid * RPT, RPT)],
                            out_hbm.at[cid].at[pl.ds(sid * RPT, RPT)])

        @pl.when(sid == NSUB - 1)
        def _():
            pltpu.sync_copy(acc_sh.at[pl.ds((NSUB - 1) * RPT, RLAST)],
                            out_hbm.at[cid].at[pl.ds((NSUB - 1) * RPT, RLAST)])

    return k(dst_all, z16)


def _segsum(hs, srcf, dst3, z128):
    mesh = plsc.VectorSubcoreMesh(core_axis_name="c", subcore_axis_name="s")

    @functools.partial(
        pl.kernel, mesh=mesh,
        out_type=jax.ShapeDtypeStruct((2, N, 128), jnp.float32),
        scratch_types=[
            pltpu.VMEM((CPT * CH,), jnp.int32),
            pltpu.VMEM((CPT, CH), jnp.int32),
            pltpu.VMEM((CH, 128), jnp.float32),
            pltpu.VMEM((CH, 128), jnp.float32),
            pltpu.VMEM_SHARED((NPAD, 128), jnp.float32),
            pltpu.SemaphoreType.DMA,
            pltpu.SemaphoreType.DMA,
        ])
    def k(hs_hbm, src_hbm, dst_hbm, zer_hbm, out_hbm,
          idxs_v, idxd_v, bufa_v, bufb_v, acc_sh, sema, semb):
        cid = lax.axis_index("c")
        sid = lax.axis_index("s")
        pltpu.sync_copy(zer_hbm.at[sid], acc_sh.at[pl.ds(sid * RPT, RPT)])
        pltpu.sync_copy(src_hbm.at[sid], idxs_v)
        pltpu.sync_copy(dst_hbm.at[sid], idxd_v)
        plsc.subcore_barrier()

        def gat(j, buf, sem):
            return pltpu.make_async_copy(
                hs_hbm.at[cid].at[idxs_v.at[pl.ds(j * CH, CH)]], buf, sem)

        gat(0, bufa_v, sema).start()

        @pl.loop(0, (CPT - 1) // 2)
        def _(p):
            j0 = 2 * p
            gat(j0 + 1, bufb_v, semb).start()
            gat(j0, bufa_v, sema).wait()
            pltpu.sync_copy(bufa_v, acc_sh.at[idxd_v.at[j0]], add=True)
            gat(j0 + 2, bufa_v, sema).start()
            gat(j0 + 1, bufb_v, semb).wait()
            pltpu.sync_copy(bufb_v, acc_sh.at[idxd_v.at[j0 + 1]], add=True)

        gat(CPT - 1, bufa_v, sema).wait()
        pltpu.sync_copy(bufa_v, acc_sh.at[idxd_v.at[CPT - 1]], add=True)
        plsc.subcore_barrier()

        @pl.when(sid < NSUB - 1)
        def _():
            pltpu.sync_copy(acc_sh.at[pl.ds(sid * RPT, RPT)],
                            out_hbm.at[cid].at[pl.ds(sid * RPT, RPT)])

        @pl.when(sid == NSUB - 1)
        def _():
            pltpu.sync_copy(acc_sh.at[pl.ds((NSUB - 1) * RPT, RLAST)],
                            out_hbm.at[cid].at[pl.ds((NSUB - 1) * RPT, RLAST)])

    return k(hs, srcf, dst3, z128)


def kernel(x, pos, aa, ei_bb, ei_ct, ln_g, ln_b,
           pw0, pb0, pw1, pb1, pw2, pb2, pw3, pb3,
           iw0, ib0, iw1, ib1,
           c0bb_wl, c0bb_wr, c0bb_b, c0ct_wl, c0ct_wr, c0ct_b,
           gn0_g, gn0_b, gn0_a,
           c1bb_wl, c1bb_wr, c1bb_b, c1ct_wl, c1ct_wr, c1ct_b,
           gn1_g, gn1_b, gn1_a,
           hw0, hb0, hw1, hb1,
           ow0, ob0, ow1, ob1, ow2, ob2, cb):
    r1 = lambda v: v.reshape(1, -1)
    src_bb = ei_bb[0].reshape(NSUB, CPT * CH)
    dst_bb = ei_bb[1].reshape(NSUB, CPT, CH)
    src_ct = ei_ct[0].reshape(NSUB, CPT * CH)
    dst_ct = ei_ct[1].reshape(NSUB, CPT, CH)
    dst_all = jnp.stack([ei_bb[1].reshape(NSUB, CPT, CH),
                         ei_ct[1].reshape(NSUB, CPT, CH)])
    z16 = jnp.zeros((NSUB, RPT, 16), jnp.float32)
    z128 = jnp.zeros((NSUB, RPT, 128), jnp.float32)

    cnts = _counts(dst_all, z16)
    cnt_bb = cnts[0]
    cnt_ct = cnts[1]

    h0s = _pre(x, pos, r1(ln_g), r1(ln_b),
               [pw0, pw1, pw2, pw3], [r1(pb0), r1(pb1), r1(pb2), r1(pb3)],
               iw0[:256], iw0[256:], r1(ib0), iw1, r1(ib1))

    s1bb = _segsum(h0s, src_bb, dst_bb, z128)
    s1ct = _segsum(h0s, src_ct, dst_ct, z128)
    h1s = _combine_gn(h0s, s1bb, s1ct, cnt_bb, cnt_ct,
                      c0bb_wl[:128], c0bb_wl[128:],
                      c0bb_wr[:128], c0bb_wr[128:], r1(c0bb_b),
                      c0ct_wl[:128], c0ct_wl[128:],
                      c0ct_wr[:128], c0ct_wr[128:], r1(c0ct_b),
                      r1(gn0_g), r1(gn0_b), r1(gn0_a))

    s2bb = _segsum(h1s, src_bb, dst_bb, z128)
    s2ct = _segsum(h1s, src_ct, dst_ct, z128)
    return _head(h1s, s2bb, s2ct, cnt_bb, cnt_ct,
                 c1bb_wl[:128], c1bb_wl[128:],
                 c1bb_wr[:128], c1bb_wr[128:], r1(c1bb_b),
                 c1ct_wl[:128], c1ct_wl[128:],
                 c1ct_wr[:128], c1ct_wr[128:], r1(c1ct_b),
                 r1(gn1_g), r1(gn1_b), r1(gn1_a),
                 hw0[:128], hw0[128:256], hw0[256:], r1(hb0), hw1, r1(hb1),
                 aa, ow0[:100], ow0[100:], r1(ob0), ow1, r1(ob1),
                 ow2, r1(ob2), cb.T)
